# Initial kernel scaffold; baseline (speedup 1.0000x reference)
import jax
import jax.numpy as jnp
from jax.experimental import pallas as pl


def kernel(x, edge_index, edge_weight, batch, edge_index_neighbor, edge_weight_neighbor, batch_neighbor, W_o, a_src_o, a_dst_o, b_o, W_n, a_src_n, a_dst_n, b_n, Wf, bf):
    raise NotImplementedError("write your pallas kernel here")


# speedup vs baseline: 21.3078x; 21.3078x over previous
import functools

import jax
import jax.numpy as jnp
from jax import lax
from jax.experimental import pallas as pl
from jax.experimental.pallas import tpu as pltpu
from jax.experimental.pallas import tpu_sc as plsc

N = 10000
D = 128
G = 128
E = 320000
NE = E + N
WIN = 64
NSUB = 16
PT = 20672
EP = PT * NSUB
NWIN = PT // WIN
SS = 640
NSP = SS * NSUB
NB = 10
BR = N // NB


def _tc_first_body(x_ref, w_ref, as_ref, ad_ref,
                   h_ref, es_ref, ed_ref, esm_ref, edm_ref):
    b = pl.program_id(1)
    t = x_ref[...]
    h = jnp.dot(t, w_ref[0], preferred_element_type=jnp.float32)
    h_ref[0] = h
    es = jnp.sum(h * as_ref[0, 0][None, :], axis=1)
    ed = jnp.sum(h * ad_ref[0, 0][None, :], axis=1)
    es_ref[0, 0] = es
    ed_ref[0, 0] = ed
    em = jnp.full((1, 1, 128), jnp.max(es), jnp.float32)
    dm = jnp.full((1, 1, 128), jnp.max(ed), jnp.float32)

    @pl.when(b == 0)
    def _():
        esm_ref[...] = em
        edm_ref[...] = dm

    @pl.when(b != 0)
    def _():
        esm_ref[...] = jnp.maximum(esm_ref[...], em)
        edm_ref[...] = jnp.maximum(edm_ref[...], dm)


def _tc_layer_body(relu, acc_ref, ssum_ref, bias_ref, w_ref, as_ref, ad_ref,
                   h_ref, es_ref, ed_ref, esm_ref, edm_ref):
    b = pl.program_id(1)
    t = (acc_ref[0] / (ssum_ref[0, 0][:, None] + 1e-16)
         + bias_ref[0, 0][None, :])
    if relu:
        t = jnp.maximum(t, 0.0)
    h = jnp.dot(t, w_ref[0], preferred_element_type=jnp.float32)
    h_ref[0] = h
    es = jnp.sum(h * as_ref[0, 0][None, :], axis=1)
    ed = jnp.sum(h * ad_ref[0, 0][None, :], axis=1)
    es_ref[0, 0] = es
    ed_ref[0, 0] = ed
    em = jnp.full((1, 1, 128), jnp.max(es), jnp.float32)
    dm = jnp.full((1, 1, 128), jnp.max(ed), jnp.float32)

    @pl.when(b == 0)
    def _():
        esm_ref[...] = em
        edm_ref[...] = dm

    @pl.when(b != 0)
    def _():
        esm_ref[...] = jnp.maximum(esm_ref[...], em)
        edm_ref[...] = jnp.maximum(edm_ref[...], dm)


_TC_OUTS = (
    jax.ShapeDtypeStruct((2, N, D), jnp.float32),
    jax.ShapeDtypeStruct((2 * NB, 1, BR), jnp.float32),
    jax.ShapeDtypeStruct((2 * NB, 1, BR), jnp.float32),
    jax.ShapeDtypeStruct((2, 1, 128), jnp.float32),
    jax.ShapeDtypeStruct((2, 1, 128), jnp.float32),
)

_TC_OUT_SPECS = [
    pl.BlockSpec((1, BR, D), lambda s, b: (s, b, 0)),
    pl.BlockSpec((1, 1, BR), lambda s, b: (s * NB + b, 0, 0)),
    pl.BlockSpec((1, 1, BR), lambda s, b: (s * NB + b, 0, 0)),
    pl.BlockSpec((1, 1, 128), lambda s, b: (s, 0, 0)),
    pl.BlockSpec((1, 1, 128), lambda s, b: (s, 0, 0)),
]


def _tc_first(x, W, a_s, a_d):
    return pl.pallas_call(
        _tc_first_body,
        grid=(2, NB),
        in_specs=[
            pl.BlockSpec((BR, D), lambda s, b: (b, 0)),
            pl.BlockSpec((1, D, D), lambda s, b: (s, 0, 0)),
            pl.BlockSpec((1, 1, D), lambda s, b: (s, 0, 0)),
            pl.BlockSpec((1, 1, D), lambda s, b: (s, 0, 0)),
        ],
        out_specs=_TC_OUT_SPECS,
        out_shape=_TC_OUTS,
    )(x, W, a_s, a_d)


def _tc_layer(acc, ssum, bias, W, a_s, a_d, relu):
    return pl.pallas_call(
        functools.partial(_tc_layer_body, relu),
        grid=(2, NB),
        in_specs=[
            pl.BlockSpec((1, BR, D), lambda s, b: (s, b, 0)),
            pl.BlockSpec((1, 1, BR), lambda s, b: (s * NB + b, 0, 0)),
            pl.BlockSpec((1, 1, D), lambda s, b: (s, 0, 0)),
            pl.BlockSpec((1, D, D), lambda s, b: (s, 0, 0)),
            pl.BlockSpec((1, 1, D), lambda s, b: (s, 0, 0)),
            pl.BlockSpec((1, 1, D), lambda s, b: (s, 0, 0)),
        ],
        out_specs=_TC_OUT_SPECS,
        out_shape=_TC_OUTS,
    )(acc, ssum, bias, W, a_s, a_d)


def _tc_final_body(acc_ref, ssum_ref, bias_ref, wf_ref, bf_ref, batch_ref,
                   out_ref, pool_ref, cnt_ref):
    b = pl.program_id(0)
    o0 = (acc_ref[0] / (ssum_ref[b, 0][:, None] + 1e-16)
          + bias_ref[0, 0][None, :])
    o1 = (acc_ref[1] / (ssum_ref[NB + b, 0][:, None] + 1e-16)
          + bias_ref[1, 0][None, :])
    o0 = jnp.maximum(o0, 0.0)
    o1 = jnp.maximum(o1, 0.0)
    f = (jnp.dot(o0, wf_ref[:D, :], preferred_element_type=jnp.float32)
         + jnp.dot(o1, wf_ref[D:, :], preferred_element_type=jnp.float32)
         + bf_ref[...])
    bt = batch_ref[0, 0, :]
    oh = (bt[:, None] == lax.broadcasted_iota(jnp.int32, (1, G), 1)
          ).astype(jnp.float32)
    contrib = lax.dot_general(oh, f, (((0,), (0,)), ((), ())),
                              preferred_element_type=jnp.float32)
    ones = jnp.ones_like(f)
    csum = lax.dot_general(oh, ones, (((0,), (0,)), ((), ())),
                           preferred_element_type=jnp.float32)

    @pl.when(b == 0)
    def _():
        pool_ref[...] = contrib
        cnt_ref[...] = csum

    @pl.when(b != 0)
    def _():
        pool_ref[...] += contrib
        cnt_ref[...] += csum

    @pl.when(b == NB - 1)
    def _():
        out_ref[...] = pool_ref[...] / jnp.maximum(cnt_ref[...], 1.0)


def _tc_final(acc, ssum, bias, Wf2, bf2, batch3):
    return pl.pallas_call(
        _tc_final_body,
        grid=(NB,),
        in_specs=[
            pl.BlockSpec((2, BR, D), lambda b: (0, b, 0)),
            pl.BlockSpec((2 * NB, 1, BR), lambda b: (0, 0, 0)),
            pl.BlockSpec((2, 1, D), lambda b: (0, 0, 0)),
            pl.BlockSpec((2 * D, D), lambda b: (0, 0)),
            pl.BlockSpec((1, D), lambda b: (0, 0)),
            pl.BlockSpec((1, 1, BR), lambda b: (b, 0, 0)),
        ],
        out_specs=pl.BlockSpec((G, D), lambda b: (0, 0)),
        out_shape=jax.ShapeDtypeStruct((G, D), jnp.float32),
        scratch_shapes=[
            pltpu.VMEM((G, D), jnp.float32),
            pltpu.VMEM((G, D), jnp.float32),
        ],
    )(acc, ssum, bias, Wf2, bf2, batch3)


def _sc_body(h_ref, es_ref, ed_ref, c_ref, src_ref, dst_ref, zr_ref, zv_ref,
             acc_out, ssum_out,
             es_t, ed_t, cbuf, srcg, dstl, wbuf, rows, acc_sh, ssum_sh, gsem):
    c = lax.axis_index("c")
    sid = lax.axis_index("s")

    pltpu.sync_copy(es_ref.at[pl.ds(c * N, N)], es_t)
    pltpu.sync_copy(ed_ref.at[pl.ds(c * N, N)], ed_t)
    pltpu.sync_copy(c_ref.at[c], cbuf)
    pltpu.sync_copy(zr_ref, acc_sh.at[pl.ds(sid * SS, SS)])
    pltpu.sync_copy(zv_ref, ssum_sh.at[pl.ds(sid * SS, SS)])
    plsc.subcore_barrier()

    coffs = c * N
    base = c * EP + sid * PT
    ebase = sid * PT
    cvec = cbuf[...]

    def window(w_i, _):
        off = base + w_i * WIN
        pltpu.sync_copy(src_ref.at[pl.ds(off, WIN)], srcg.at[0])
        pltpu.sync_copy(dst_ref.at[pl.ds(off, WIN)], dstl.at[0])
        pltpu.async_copy(h_ref.at[srcg.at[0]], rows.at[0], gsem).wait()
        eoff = ebase + w_i * WIN
        for j in range(WIN // 16):
            sv = srcg[0, pl.ds(j * 16, 16)]
            dv = dstl[0, pl.ds(j * 16, 16)]
            esv = plsc.load_gather(es_t, [sv - coffs])
            edv = plsc.load_gather(ed_t, [dv])
            e = esv + edv
            e = jnp.where(e > 0.0, e, 0.2 * e)
            w = jnp.exp(e - cvec)
            eid = eoff + j * 16 + lax.iota(jnp.int32, 16)
            w = jnp.where(eid < NE, w, 0.0)
            wbuf[0, pl.ds(j * 16, 16)] = w
        for g in range(WIN // 16):
            wg = wbuf[0, pl.ds(g * 16, 16)]
            for l in range(16):
                e_ = g * 16 + l
                wv = jnp.full((16,), wg[l], jnp.float32)
                for k in range(D // 16):
                    rows[0, e_, pl.ds(k * 16, 16)] = (
                        rows[0, e_, pl.ds(k * 16, 16)] * wv)
        pltpu.sync_copy(rows.at[0], acc_sh.at[dstl.at[0]], add=True)
        pltpu.sync_copy(wbuf.at[0], ssum_sh.at[dstl.at[0]], add=True)
        return 0

    lax.fori_loop(0, NWIN, window, 0)
    plsc.subcore_barrier()

    pltpu.sync_copy(acc_sh.at[pl.ds(sid * SS, SS)],
                    acc_out.at[pl.ds(c * NSP + sid * SS, SS)])
    pltpu.sync_copy(ssum_sh.at[pl.ds(sid * SS, SS)],
                    ssum_out.at[pl.ds(c * NSP + sid * SS, SS)])


@functools.lru_cache(maxsize=1)
def _sc_gat_fn():
    return pl.kernel(
        _sc_body,
        out_type=(
            jax.ShapeDtypeStruct((2 * NSP, D), jnp.float32),
            jax.ShapeDtypeStruct((2 * NSP,), jnp.float32),
        ),
        mesh=plsc.VectorSubcoreMesh(core_axis_name="c", subcore_axis_name="s",
                                    num_cores=2, num_subcores=NSUB),
        compiler_params=pltpu.CompilerParams(needs_layout_passes=False),
        scratch_types=[
            pltpu.VMEM((N,), jnp.float32),
            pltpu.VMEM((N,), jnp.float32),
            pltpu.VMEM((16,), jnp.float32),
            pltpu.VMEM((2, WIN), jnp.int32),
            pltpu.VMEM((2, WIN), jnp.int32),
            pltpu.VMEM((2, WIN), jnp.float32),
            pltpu.VMEM((2, WIN, D), jnp.float32),
            pltpu.VMEM_SHARED((NSP, D), jnp.float32),
            pltpu.VMEM_SHARED((NSP,), jnp.float32),
            pltpu.SemaphoreType.DMA,
        ],
    )


def kernel(x, edge_index, edge_weight, batch, edge_index_neighbor,
           edge_weight_neighbor, batch_neighbor, W_o, a_src_o, a_dst_o, b_o,
           W_n, a_src_n, a_dst_n, b_n, Wf, bf):
    loop = jnp.arange(N, dtype=jnp.int32)
    pad = jnp.arange(EP - NE, dtype=jnp.int32) % N

    def mk(ei, setid):
        src = jnp.concatenate([ei[0], loop, pad]) + setid * N
        dst = jnp.concatenate([ei[1], loop, pad])
        return src, dst

    src_o, dst_o = mk(edge_index, 0)
    src_n, dst_n = mk(edge_index_neighbor, 1)
    src = jnp.concatenate([src_o, src_n])
    dst = jnp.concatenate([dst_o, dst_n])

    Ws = jnp.stack([W_o, W_n], axis=1)
    As = jnp.stack([a_src_o, a_src_n], axis=1)[:, :, None, :]
    Ad = jnp.stack([a_dst_o, a_dst_n], axis=1)[:, :, None, :]
    Bs = jnp.stack([b_o, b_n], axis=1)[:, :, None, :]

    zr = jnp.zeros((SS, D), jnp.float32)
    zv = jnp.zeros((SS,), jnp.float32)
    batch3 = batch.astype(jnp.int32).reshape(NB, 1, BR)

    def cbound(esm, edm):
        raw = jnp.max(esm, axis=(1, 2)) + jnp.max(edm, axis=(1, 2))
        c2 = jnp.where(raw > 0, raw, 0.2 * raw)
        return jnp.broadcast_to(c2[:, None], (2, 16))

    h, es, ed, esm, edm = _tc_first(x, Ws[0], As[0], Ad[0])
    for i in range(3):
        c16 = cbound(esm, edm)
        acc, ssum = _sc_gat_fn()(h.reshape(2 * N, D), es.reshape(2 * N),
                                 ed.reshape(2 * N), c16, src, dst, zr, zv)
        acc = acc.reshape(2, NSP, D)[:, :N]
        ssum = ssum.reshape(2, NSP)[:, :N].reshape(2 * NB, 1, BR)
        if i < 2:
            h, es, ed, esm, edm = _tc_layer(acc, ssum, Bs[i], Ws[i + 1],
                                            As[i + 1], Ad[i + 1], relu=(i >= 1))

    return _tc_final(acc, ssum, Bs[2], Wf[2], bf[2][None, :], batch3)


# speedup vs baseline: 34.2521x; 1.6075x over previous
import functools

import jax
import jax.numpy as jnp
from jax import lax
from jax.experimental import pallas as pl
from jax.experimental.pallas import tpu as pltpu
from jax.experimental.pallas import tpu_sc as plsc

N = 10000
D = 128
G = 128
E = 320000
NE = E + N
WIN = 64
NSUB = 16
PT = 20736
EP = PT * NSUB
NWIN = PT // WIN
NPAIR = NWIN // 2
SS = 640
NSP = SS * NSUB
NB = 10
BR = N // NB


def _tc_first_body(x_ref, w_ref, as_ref, ad_ref,
                   h_ref, es_ref, ed_ref, esm_ref, edm_ref):
    b = pl.program_id(1)
    t = x_ref[...]
    h = jnp.dot(t, w_ref[0], preferred_element_type=jnp.float32)
    h_ref[0] = h
    es = jnp.sum(h * as_ref[0, 0][None, :], axis=1)
    ed = jnp.sum(h * ad_ref[0, 0][None, :], axis=1)
    es_ref[0, 0] = es
    ed_ref[0, 0] = ed
    em = jnp.full((1, 1, 128), jnp.max(es), jnp.float32)
    dm = jnp.full((1, 1, 128), jnp.max(ed), jnp.float32)

    @pl.when(b == 0)
    def _():
        esm_ref[...] = em
        edm_ref[...] = dm

    @pl.when(b != 0)
    def _():
        esm_ref[...] = jnp.maximum(esm_ref[...], em)
        edm_ref[...] = jnp.maximum(edm_ref[...], dm)


def _tc_layer_body(relu, acc_ref, ssum_ref, bias_ref, w_ref, as_ref, ad_ref,
                   h_ref, es_ref, ed_ref, esm_ref, edm_ref):
    b = pl.program_id(1)
    t = (acc_ref[0] / (ssum_ref[0, 0][:, None] + 1e-16)
         + bias_ref[0, 0][None, :])
    if relu:
        t = jnp.maximum(t, 0.0)
    h = jnp.dot(t, w_ref[0], preferred_element_type=jnp.float32)
    h_ref[0] = h
    es = jnp.sum(h * as_ref[0, 0][None, :], axis=1)
    ed = jnp.sum(h * ad_ref[0, 0][None, :], axis=1)
    es_ref[0, 0] = es
    ed_ref[0, 0] = ed
    em = jnp.full((1, 1, 128), jnp.max(es), jnp.float32)
    dm = jnp.full((1, 1, 128), jnp.max(ed), jnp.float32)

    @pl.when(b == 0)
    def _():
        esm_ref[...] = em
        edm_ref[...] = dm

    @pl.when(b != 0)
    def _():
        esm_ref[...] = jnp.maximum(esm_ref[...], em)
        edm_ref[...] = jnp.maximum(edm_ref[...], dm)


_TC_OUTS = (
    jax.ShapeDtypeStruct((2, N, D), jnp.float32),
    jax.ShapeDtypeStruct((2 * NB, 1, BR), jnp.float32),
    jax.ShapeDtypeStruct((2 * NB, 1, BR), jnp.float32),
    jax.ShapeDtypeStruct((2, 1, 128), jnp.float32),
    jax.ShapeDtypeStruct((2, 1, 128), jnp.float32),
)

_TC_OUT_SPECS = [
    pl.BlockSpec((1, BR, D), lambda s, b: (s, b, 0)),
    pl.BlockSpec((1, 1, BR), lambda s, b: (s * NB + b, 0, 0)),
    pl.BlockSpec((1, 1, BR), lambda s, b: (s * NB + b, 0, 0)),
    pl.BlockSpec((1, 1, 128), lambda s, b: (s, 0, 0)),
    pl.BlockSpec((1, 1, 128), lambda s, b: (s, 0, 0)),
]


def _tc_first(x, W, a_s, a_d):
    return pl.pallas_call(
        _tc_first_body,
        grid=(2, NB),
        in_specs=[
            pl.BlockSpec((BR, D), lambda s, b: (b, 0)),
            pl.BlockSpec((1, D, D), lambda s, b: (s, 0, 0)),
            pl.BlockSpec((1, 1, D), lambda s, b: (s, 0, 0)),
            pl.BlockSpec((1, 1, D), lambda s, b: (s, 0, 0)),
        ],
        out_specs=_TC_OUT_SPECS,
        out_shape=_TC_OUTS,
    )(x, W, a_s, a_d)


def _tc_layer(acc, ssum, bias, W, a_s, a_d, relu):
    return pl.pallas_call(
        functools.partial(_tc_layer_body, relu),
        grid=(2, NB),
        in_specs=[
            pl.BlockSpec((1, BR, D), lambda s, b: (s, b, 0)),
            pl.BlockSpec((1, 1, BR), lambda s, b: (s * NB + b, 0, 0)),
            pl.BlockSpec((1, 1, D), lambda s, b: (s, 0, 0)),
            pl.BlockSpec((1, D, D), lambda s, b: (s, 0, 0)),
            pl.BlockSpec((1, 1, D), lambda s, b: (s, 0, 0)),
            pl.BlockSpec((1, 1, D), lambda s, b: (s, 0, 0)),
        ],
        out_specs=_TC_OUT_SPECS,
        out_shape=_TC_OUTS,
    )(acc, ssum, bias, W, a_s, a_d)


def _tc_final_body(acc_ref, ssum_ref, bias_ref, wf_ref, bf_ref, batch_ref,
                   out_ref, pool_ref, cnt_ref):
    b = pl.program_id(0)
    o0 = (acc_ref[0] / (ssum_ref[b, 0][:, None] + 1e-16)
          + bias_ref[0, 0][None, :])
    o1 = (acc_ref[1] / (ssum_ref[NB + b, 0][:, None] + 1e-16)
          + bias_ref[1, 0][None, :])
    o0 = jnp.maximum(o0, 0.0)
    o1 = jnp.maximum(o1, 0.0)
    f = (jnp.dot(o0, wf_ref[:D, :], preferred_element_type=jnp.float32)
         + jnp.dot(o1, wf_ref[D:, :], preferred_element_type=jnp.float32)
         + bf_ref[...])
    bt = batch_ref[0, 0, :]
    oh = (bt[:, None] == lax.broadcasted_iota(jnp.int32, (1, G), 1)
          ).astype(jnp.float32)
    contrib = lax.dot_general(oh, f, (((0,), (0,)), ((), ())),
                              preferred_element_type=jnp.float32)
    ones = jnp.ones_like(f)
    csum = lax.dot_general(oh, ones, (((0,), (0,)), ((), ())),
                           preferred_element_type=jnp.float32)

    @pl.when(b == 0)
    def _():
        pool_ref[...] = contrib
        cnt_ref[...] = csum

    @pl.when(b != 0)
    def _():
        pool_ref[...] += contrib
        cnt_ref[...] += csum

    @pl.when(b == NB - 1)
    def _():
        out_ref[...] = pool_ref[...] / jnp.maximum(cnt_ref[...], 1.0)


def _tc_final(acc, ssum, bias, Wf2, bf2, batch3):
    return pl.pallas_call(
        _tc_final_body,
        grid=(NB,),
        in_specs=[
            pl.BlockSpec((2, BR, D), lambda b: (0, b, 0)),
            pl.BlockSpec((2 * NB, 1, BR), lambda b: (0, 0, 0)),
            pl.BlockSpec((2, 1, D), lambda b: (0, 0, 0)),
            pl.BlockSpec((2 * D, D), lambda b: (0, 0)),
            pl.BlockSpec((1, D), lambda b: (0, 0)),
            pl.BlockSpec((1, 1, BR), lambda b: (b, 0, 0)),
        ],
        out_specs=pl.BlockSpec((G, D), lambda b: (0, 0)),
        out_shape=jax.ShapeDtypeStruct((G, D), jnp.float32),
        scratch_shapes=[
            pltpu.VMEM((G, D), jnp.float32),
            pltpu.VMEM((G, D), jnp.float32),
        ],
    )(acc, ssum, bias, Wf2, bf2, batch3)


def _sc_body(h_ref, es_ref, ed_ref, c_ref, src_ref, dst_ref, zr_ref, zv_ref,
             acc_out, ssum_out,
             es_t, ed_t, cbuf, srcg, dstl, wbuf, rows, acc_sh, ssum_sh,
             gsem0, gsem1, asem0, asem1, bsem0, bsem1):
    c = lax.axis_index("c")
    sid = lax.axis_index("s")

    pltpu.sync_copy(es_ref.at[pl.ds(c * N, N)], es_t)
    pltpu.sync_copy(ed_ref.at[pl.ds(c * N, N)], ed_t)
    pltpu.sync_copy(c_ref.at[c], cbuf)
    pltpu.sync_copy(zr_ref, acc_sh.at[pl.ds(sid * SS, SS)])
    pltpu.sync_copy(zv_ref, ssum_sh.at[pl.ds(sid * SS, SS)])
    plsc.subcore_barrier()

    coffs = c * N
    base = c * EP + sid * PT
    ebase = sid * PT
    cvec = cbuf[...]
    gsem = (gsem0, gsem1)
    asem = (asem0, asem1)
    bsem = (bsem0, bsem1)

    def load_idx(w, p):
        off = base + w * WIN
        pltpu.sync_copy(src_ref.at[pl.ds(off, WIN)], srcg.at[p])
        pltpu.sync_copy(dst_ref.at[pl.ds(off, WIN)], dstl.at[p])

    def compute_w(w, p):
        eoff = ebase + w * WIN
        for j in range(WIN // 16):
            sv = srcg[p, pl.ds(j * 16, 16)]
            dv = dstl[p, pl.ds(j * 16, 16)]
            esv = plsc.load_gather(es_t, [sv - coffs])
            edv = plsc.load_gather(ed_t, [dv])
            e = esv + edv
            e = jnp.where(e > 0.0, e, 0.2 * e)
            wv = jnp.exp(e - cvec)
            eid = eoff + j * 16 + lax.iota(jnp.int32, 16)
            wbuf[p, pl.ds(j * 16, 16)] = jnp.where(eid < NE, wv, 0.0)

    def scale(p):
        for g in range(WIN // 16):
            wg = wbuf[p, pl.ds(g * 16, 16)]
            for l in range(16):
                e_ = g * 16 + l
                wv = jnp.full((16,), wg[l], jnp.float32)
                for k in range(D // 16):
                    rows[p, e_, pl.ds(k * 16, 16)] = (
                        rows[p, e_, pl.ds(k * 16, 16)] * wv)

    for p in (0, 1):
        load_idx(p, p)
        pltpu.async_copy(h_ref.at[srcg.at[p]], rows.at[p], gsem[p])

    def pair(i, _):
        for p in (0, 1):
            w = 2 * i + p
            compute_w(w, p)
            pltpu.make_async_copy(h_ref.at[srcg.at[p]], rows.at[p],
                                  gsem[p]).wait()
            scale(p)
            pltpu.async_copy(rows.at[p], acc_sh.at[dstl.at[p]], asem[p],
                             add=True)
            pltpu.async_copy(wbuf.at[p], ssum_sh.at[dstl.at[p]], bsem[p],
                             add=True)

        @pl.when(i < NPAIR - 1)
        def _():
            for p in (0, 1):
                pltpu.make_async_copy(rows.at[p], acc_sh.at[dstl.at[p]],
                                      asem[p]).wait()
                pltpu.make_async_copy(wbuf.at[p], ssum_sh.at[dstl.at[p]],
                                      bsem[p]).wait()
                load_idx(2 * i + 2 + p, p)
                pltpu.async_copy(h_ref.at[srcg.at[p]], rows.at[p], gsem[p])

        return 0

    lax.fori_loop(0, NPAIR, pair, 0)
    for p in (0, 1):
        pltpu.make_async_copy(rows.at[p], acc_sh.at[dstl.at[p]],
                              asem[p]).wait()
        pltpu.make_async_copy(wbuf.at[p], ssum_sh.at[dstl.at[p]],
                              bsem[p]).wait()
    plsc.subcore_barrier()

    pltpu.sync_copy(acc_sh.at[pl.ds(sid * SS, SS)],
                    acc_out.at[pl.ds(c * NSP + sid * SS, SS)])
    pltpu.sync_copy(ssum_sh.at[pl.ds(sid * SS, SS)],
                    ssum_out.at[pl.ds(c * NSP + sid * SS, SS)])


@functools.lru_cache(maxsize=1)
def _sc_gat_fn():
    return pl.kernel(
        _sc_body,
        out_type=(
            jax.ShapeDtypeStruct((2 * NSP, D), jnp.float32),
            jax.ShapeDtypeStruct((2 * NSP,), jnp.float32),
        ),
        mesh=plsc.VectorSubcoreMesh(core_axis_name="c", subcore_axis_name="s",
                                    num_cores=2, num_subcores=NSUB),
        compiler_params=pltpu.CompilerParams(needs_layout_passes=False),
        scratch_types=[
            pltpu.VMEM((N,), jnp.float32),
            pltpu.VMEM((N,), jnp.float32),
            pltpu.VMEM((16,), jnp.float32),
            pltpu.VMEM((2, WIN), jnp.int32),
            pltpu.VMEM((2, WIN), jnp.int32),
            pltpu.VMEM((2, WIN), jnp.float32),
            pltpu.VMEM((2, WIN, D), jnp.float32),
            pltpu.VMEM_SHARED((NSP, D), jnp.float32),
            pltpu.VMEM_SHARED((NSP,), jnp.float32),
            pltpu.SemaphoreType.DMA,
            pltpu.SemaphoreType.DMA,
            pltpu.SemaphoreType.DMA,
            pltpu.SemaphoreType.DMA,
            pltpu.SemaphoreType.DMA,
            pltpu.SemaphoreType.DMA,
        ],
    )


def kernel(x, edge_index, edge_weight, batch, edge_index_neighbor,
           edge_weight_neighbor, batch_neighbor, W_o, a_src_o, a_dst_o, b_o,
           W_n, a_src_n, a_dst_n, b_n, Wf, bf):
    loop = jnp.arange(N, dtype=jnp.int32)
    pad = jnp.arange(EP - NE, dtype=jnp.int32) % N

    def mk(ei, setid):
        src = jnp.concatenate([ei[0], loop, pad]) + setid * N
        dst = jnp.concatenate([ei[1], loop, pad])
        return src, dst

    src_o, dst_o = mk(edge_index, 0)
    src_n, dst_n = mk(edge_index_neighbor, 1)
    src = jnp.concatenate([src_o, src_n])
    dst = jnp.concatenate([dst_o, dst_n])

    Ws = jnp.stack([W_o, W_n], axis=1)
    As = jnp.stack([a_src_o, a_src_n], axis=1)[:, :, None, :]
    Ad = jnp.stack([a_dst_o, a_dst_n], axis=1)[:, :, None, :]
    Bs = jnp.stack([b_o, b_n], axis=1)[:, :, None, :]

    zr = jnp.zeros((SS, D), jnp.float32)
    zv = jnp.zeros((SS,), jnp.float32)
    batch3 = batch.astype(jnp.int32).reshape(NB, 1, BR)

    def cbound(esm, edm):
        raw = jnp.max(esm, axis=(1, 2)) + jnp.max(edm, axis=(1, 2))
        c2 = jnp.where(raw > 0, raw, 0.2 * raw)
        return jnp.broadcast_to(c2[:, None], (2, 16))

    h, es, ed, esm, edm = _tc_first(x, Ws[0], As[0], Ad[0])
    for i in range(3):
        c16 = cbound(esm, edm)
        acc, ssum = _sc_gat_fn()(h.reshape(2 * N, D), es.reshape(2 * N),
                                 ed.reshape(2 * N), c16, src, dst, zr, zv)
        acc = acc.reshape(2, NSP, D)[:, :N]
        ssum = ssum.reshape(2, NSP)[:, :N].reshape(2 * NB, 1, BR)
        if i < 2:
            h, es, ed, esm, edm = _tc_layer(acc, ssum, Bs[i], Ws[i + 1],
                                            As[i + 1], Ad[i + 1], relu=(i >= 1))

    return _tc_final(acc, ssum, Bs[2], Wf[2], bf[2][None, :], batch3)
